# baseline (speedup 1.0000x reference)
import functools

import jax
import jax.numpy as jnp
from jax import lax
from jax.experimental import pallas as pl
from jax.experimental.pallas import tpu as pltpu
from jax.experimental.pallas import tpu_sc as plsc

NN = 10000
DD = 128
GG = 64
EE = 160000
NP = 10240
NR = NP // 128
NC, NS = 2, 16
NW = NC * NS
CH = 8
NCH = 10
TSTEPS = NCH * CH
EP = NW * TSTEPS * 128
TRASH = NN
RB = 1024
GRID = NP // RB
RPT = NP // NS
GPT = GG // NW
WPOOL = 64

_Z16 = functools.partial(jnp.zeros, (16,), jnp.float32)


def _deg_body(dst_hbm, out_hbm, didx, ones, zbuf, scnt, sem):
    cid = lax.axis_index("c")
    sid = lax.axis_index("s")
    wid = sid * NC + cid
    pltpu.sync_copy(dst_hbm.at[wid], didx)

    def _fill_ones(i, carry):
        ones[pl.ds(i * 16, 16)] = jnp.ones((16,), jnp.float32)
        return carry
    lax.fori_loop(0, 8, _fill_ones, 0)

    def _fill_zero(i, carry):
        zbuf[pl.ds(i * 16, 16)] = _Z16()
        return carry
    lax.fori_loop(0, 64, _fill_zero, 0)
    pltpu.sync_copy(zbuf, scnt.at[pl.ds(sid * 1024, 1024)])
    plsc.subcore_barrier()

    def _chunk(p, carry):
        for k in range(CH):
            pltpu.async_copy(ones, scnt.at[didx.at[p, k]], sem, add=True)
        for k in range(CH):
            pltpu.make_async_copy(ones, scnt.at[didx.at[0, 0]], sem).wait()
        return carry
    lax.fori_loop(0, NCH, _chunk, 0)
    plsc.subcore_barrier()
    pltpu.sync_copy(scnt.at[pl.ds(sid * 1024, 1024)],
                    out_hbm.at[cid, pl.ds(sid * 1024, 1024)])


def _scatter_body(y_hbm, src_hbm, dst_hbm, out_hbm,
                  sidx, didx, rows, acc, sem0, sem1, isem):
    cid = lax.axis_index("c")
    sid = lax.axis_index("s")
    wid = sid * NC + cid
    pltpu.async_copy(src_hbm.at[wid, 0], sidx.at[0], isem)
    pltpu.async_copy(dst_hbm.at[wid, 0], didx.at[0], isem)

    def _zero_rows(i, carry):
        for k in range(8):
            rows[0, i, pl.ds(k * 16, 16)] = _Z16()
        return carry
    lax.fori_loop(0, 128, _zero_rows, 0)
    base = sid * RPT
    for k in range(RPT // 128):
        pltpu.sync_copy(rows.at[0], acc.at[pl.ds(base + k * 128, 128)])
    plsc.subcore_barrier()

    def _chunk(c, carry):
        slot = lax.rem(c, 2)
        pltpu.make_async_copy(src_hbm.at[wid, 0], sidx.at[0], isem).wait()
        pltpu.make_async_copy(dst_hbm.at[wid, 0], didx.at[0], isem).wait()

        @pl.when(c + 1 < NCH)
        def _():
            pltpu.async_copy(src_hbm.at[wid, c + 1], sidx.at[1 - slot], isem)
            pltpu.async_copy(dst_hbm.at[wid, c + 1], didx.at[1 - slot], isem)

        pltpu.async_copy(y_hbm.at[sidx.at[slot, 0]], rows.at[0], sem0)
        for k in range(CH):
            r = k % 2
            sem = sem0 if r == 0 else sem1
            if k + 1 < CH:
                nsem = sem1 if r == 0 else sem0
                pltpu.async_copy(
                    y_hbm.at[sidx.at[slot, k + 1]], rows.at[1 - r], nsem)
            pltpu.make_async_copy(y_hbm.at[sidx.at[slot, k]], rows.at[r], sem).wait()
            pltpu.sync_copy(rows.at[r], acc.at[didx.at[slot, k]], add=True)
        return carry
    lax.fori_loop(0, NCH, _chunk, 0)
    plsc.subcore_barrier()
    pltpu.sync_copy(acc.at[pl.ds(base, RPT)], out_hbm.at[cid, pl.ds(base, RPT)])


def _pool_body(h3_hbm, se_hbm, out_hbm, sebuf, win, accb):
    cid = lax.axis_index("c")
    sid = lax.axis_index("s")
    wid = sid * NC + cid
    pltpu.sync_copy(se_hbm, sebuf)
    for gi in range(GPT):
        for k in range(8):
            accb[gi, pl.ds(k * 16, 16)] = _Z16()
    for gi in range(GPT):
        g = wid * GPT + gi
        sev = sebuf[pl.ds(g, 16)]
        s = sev[0]
        e = sev[1]
        s0 = (s // 8) * 8

        def _window(w, carry):
            base = pl.multiple_of(s0 + w * WPOOL, 8)
            pltpu.sync_copy(h3_hbm.at[pl.ds(base, WPOOL)], win)

            def _row(j, c2):
                @pl.when(jnp.logical_and(base + j >= s, base + j < e))
                def _():
                    for k in range(8):
                        accb[gi, pl.ds(k * 16, 16)] = (
                            accb[gi, pl.ds(k * 16, 16)]
                            + win[j, pl.ds(k * 16, 16)])
                return c2
            lax.fori_loop(0, WPOOL, _row, 0, unroll=False)
            return carry
        lax.fori_loop(0, lax.div(e - s0 + (WPOOL - 1), WPOOL), _window, 0,
                      unroll=False)
    pltpu.sync_copy(accb, out_hbm.at[wid, pl.ds(0, GPT)])


@functools.lru_cache(maxsize=None)
def _sc_kernels():
    mesh = plsc.VectorSubcoreMesh(
        core_axis_name="c", subcore_axis_name="s",
        num_cores=NC, num_subcores=NS)
    deg = pl.kernel(
        _deg_body,
        out_type=jax.ShapeDtypeStruct((NC, 16384), jnp.float32),
        mesh=mesh,
        scratch_types=[
            pltpu.VMEM((NCH, CH, 128), jnp.int32),
            pltpu.VMEM((128,), jnp.float32),
            pltpu.VMEM((1024,), jnp.float32),
            pltpu.VMEM_SHARED((16384,), jnp.float32),
            pltpu.SemaphoreType.DMA,
        ],
    )
    scatter = pl.kernel(
        _scatter_body,
        out_type=jax.ShapeDtypeStruct((NC, NP, DD), jnp.float32),
        mesh=mesh,
        scratch_types=[
            pltpu.VMEM((2, CH, 128), jnp.int32),
            pltpu.VMEM((2, CH, 128), jnp.int32),
            pltpu.VMEM((2, 128, DD), jnp.float32),
            pltpu.VMEM_SHARED((NP, DD), jnp.float32),
            pltpu.SemaphoreType.DMA,
            pltpu.SemaphoreType.DMA,
            pltpu.SemaphoreType.DMA,
        ],
    )
    pool = pl.kernel(
        _pool_body,
        out_type=jax.ShapeDtypeStruct((NW, 8, DD), jnp.float32),
        mesh=mesh,
        scratch_types=[
            pltpu.VMEM((128,), jnp.int32),
            pltpu.VMEM((WPOOL, DD), jnp.float32),
            pltpu.VMEM((GPT, DD), jnp.float32),
        ],
    )
    return deg, scatter, pool


def _scale_rows(v, dinv):
    return (v.reshape(RB // 128, 128, DD) * dinv[:, :, None]).reshape(RB, DD)


def _tc_pre_body(cnt_ref, x_ref, w_ref, dinv_ref, y_ref):
    deg = jnp.sum(cnt_ref[...], axis=0) + 1.0
    dinv = lax.rsqrt(deg)
    dinv_ref[...] = dinv
    xw = jnp.dot(x_ref[...], w_ref[...], preferred_element_type=jnp.float32)
    y_ref[...] = _scale_rows(xw, dinv)


_pre_call = pl.pallas_call(
    _tc_pre_body,
    grid=(GRID,),
    in_specs=[
        pl.BlockSpec((NC, RB // 128, 128), lambda i: (0, i, 0)),
        pl.BlockSpec((RB, DD), lambda i: (i, 0)),
        pl.BlockSpec((DD, DD), lambda i: (0, 0)),
    ],
    out_specs=[
        pl.BlockSpec((RB // 128, 128), lambda i: (i, 0)),
        pl.BlockSpec((RB, DD), lambda i: (i, 0)),
    ],
    out_shape=[
        jax.ShapeDtypeStruct((NR, 128), jnp.float32),
        jax.ShapeDtypeStruct((NP, DD), jnp.float32),
    ],
)


def _tc_mid_body(acc_ref, y_ref, dinv_ref, b_ref, w_ref, ynext_ref):
    dinv = dinv_ref[...]
    s = acc_ref[0] + acc_ref[1] + y_ref[...]
    h = jnp.maximum(_scale_rows(s, dinv) + b_ref[...], 0.0)
    hw = jnp.dot(h, w_ref[...], preferred_element_type=jnp.float32)
    ynext_ref[...] = _scale_rows(hw, dinv)


_mid_call = pl.pallas_call(
    _tc_mid_body,
    grid=(GRID,),
    in_specs=[
        pl.BlockSpec((2, RB, DD), lambda i: (0, i, 0)),
        pl.BlockSpec((RB, DD), lambda i: (i, 0)),
        pl.BlockSpec((RB // 128, 128), lambda i: (i, 0)),
        pl.BlockSpec((1, DD), lambda i: (0, 0)),
        pl.BlockSpec((DD, DD), lambda i: (0, 0)),
    ],
    out_specs=pl.BlockSpec((RB, DD), lambda i: (i, 0)),
    out_shape=jax.ShapeDtypeStruct((NP, DD), jnp.float32),
)


def _bn(z, g, b):
    mu = jnp.mean(z, axis=0, keepdims=True)
    var = jnp.mean((z - mu) ** 2, axis=0, keepdims=True)
    return (z - mu) / jnp.sqrt(var + 1e-5) * g + b


def _lrelu(z):
    return jnp.where(z > 0, z, 0.01 * z)


def _tc_h3_body(acc_ref, y_ref, dinv_ref, b_ref, batch_ref,
                h3_ref, se_ref, cntg):
    i = pl.program_id(0)

    @pl.when(i == 0)
    def _():
        cntg[...] = jnp.zeros((1, GG), jnp.float32)

    dinv = dinv_ref[...]
    s = acc_ref[0] + acc_ref[1] + y_ref[...]
    h3_ref[...] = _scale_rows(s, dinv) + b_ref[...]
    bt = batch_ref[...]
    gid = lax.broadcasted_iota(jnp.int32, (1, 1, GG), 2)
    pt = (bt[:, :, None] == gid).astype(jnp.float32).reshape(RB, GG)
    ones_row = jnp.ones((1, RB), jnp.float32)
    cntg[...] += jnp.dot(ones_row, pt, preferred_element_type=jnp.float32,
                         precision=lax.Precision.HIGHEST)

    @pl.when(i == GRID - 1)
    def _():
        gi2 = lax.broadcasted_iota(jnp.int32, (GG, 128), 0)
        gj2 = lax.broadcasted_iota(jnp.int32, (GG, 128), 1)
        ut = (gi2 < gj2).astype(jnp.float32)
        starts = jnp.dot(cntg[...], ut, preferred_element_type=jnp.float32,
                         precision=lax.Precision.HIGHEST)
        se_ref[...] = starts.astype(jnp.int32)


_h3_call = pl.pallas_call(
    _tc_h3_body,
    grid=(GRID,),
    in_specs=[
        pl.BlockSpec((2, RB, DD), lambda i: (0, i, 0)),
        pl.BlockSpec((RB, DD), lambda i: (i, 0)),
        pl.BlockSpec((RB // 128, 128), lambda i: (i, 0)),
        pl.BlockSpec((1, DD), lambda i: (0, 0)),
        pl.BlockSpec((RB // 128, 128), lambda i: (i, 0)),
    ],
    out_specs=[
        pl.BlockSpec((RB, DD), lambda i: (i, 0)),
        pl.BlockSpec((1, 128), lambda i: (0, 0)),
    ],
    out_shape=[
        jax.ShapeDtypeStruct((NP, DD), jnp.float32),
        jax.ShapeDtypeStruct((1, 128), jnp.int32),
    ],
    scratch_shapes=[pltpu.VMEM((1, GG), jnp.float32)],
)


def _tc_head_body(p_ref, fw1, fb1, g1, be1, fw2, fb2, g2, be2, fw3, fb3,
                  out_ref):
    p = p_ref[...]
    z = jnp.dot(p, fw1[...], preferred_element_type=jnp.float32) + fb1[...]
    z = _bn(_lrelu(z), g1[...], be1[...])
    z = jnp.dot(z, fw2[...], preferred_element_type=jnp.float32) + fb2[...]
    z = _bn(_lrelu(z), g2[...], be2[...])
    out_ref[...] = jnp.dot(z, fw3[...].reshape(DD, 1),
                           preferred_element_type=jnp.float32) + fb3[...]


_head_call = pl.pallas_call(
    _tc_head_body,
    in_specs=[pl.BlockSpec((GG, DD), lambda: (0, 0)),
              pl.BlockSpec((DD, DD), lambda: (0, 0)),
              pl.BlockSpec((1, DD), lambda: (0, 0)),
              pl.BlockSpec((1, DD), lambda: (0, 0)),
              pl.BlockSpec((1, DD), lambda: (0, 0)),
              pl.BlockSpec((DD, DD), lambda: (0, 0)),
              pl.BlockSpec((1, DD), lambda: (0, 0)),
              pl.BlockSpec((1, DD), lambda: (0, 0)),
              pl.BlockSpec((1, DD), lambda: (0, 0)),
              pl.BlockSpec((1, DD), lambda: (0, 0)),
              pl.BlockSpec((1, 1), lambda: (0, 0))],
    out_specs=pl.BlockSpec((GG, 1), lambda: (0, 0)),
    out_shape=jax.ShapeDtypeStruct((GG, 1), jnp.float32),
)


def kernel(x, edge_index_intra, edge_index_inter, batch,
           W1, b1, W2, b2, W3, b3,
           fW1, fb1, g1, be1, fW2, fb2, g2, be2, fW3, fb3):
    src = jnp.concatenate([edge_index_intra[0], edge_index_inter[0]])
    dst = jnp.concatenate([edge_index_intra[1], edge_index_inter[1]])
    pad = jnp.full((EP - 2 * EE,), TRASH, jnp.int32)
    srcp = jnp.concatenate([src, pad]).reshape(NW, NCH, CH, 128)
    dstp = jnp.concatenate([dst, pad]).reshape(NW, NCH, CH, 128)
    xp = jnp.pad(x, ((0, NP - NN), (0, 0)))
    batchp = jnp.pad(batch, (0, NP - NN), constant_values=GG).reshape(NR, 128)

    deg_call, scatter_call, pool_call = _sc_kernels()
    cnt = deg_call(dstp)
    cnt2 = cnt[:, :NP].reshape(NC, NR, 128)

    dinv, y1 = _pre_call(cnt2, xp, W1)
    acc1 = scatter_call(y1, srcp, dstp)
    y2 = _mid_call(acc1, y1, dinv, b1.reshape(1, DD), W2)
    acc2 = scatter_call(y2, srcp, dstp)
    y3 = _mid_call(acc2, y2, dinv, b2.reshape(1, DD), W3)
    acc3 = scatter_call(y3, srcp, dstp)

    h3, se = _h3_call(acc3, y3, dinv, b3.reshape(1, DD), batchp)
    pooled = pool_call(h3, se.reshape(128))
    out = _head_call(
        pooled[:, :GPT].reshape(GG, DD),
        fW1, fb1.reshape(1, DD), g1.reshape(1, DD), be1.reshape(1, DD),
        fW2, fb2.reshape(1, DD), g2.reshape(1, DD), be2.reshape(1, DD),
        fW3.reshape(1, DD), fb3.reshape(1, 1))
    return out.reshape(-1)


# speedup vs baseline: 3.8455x; 3.8455x over previous
import functools

import jax
import jax.numpy as jnp
from jax import lax
from jax.experimental import pallas as pl
from jax.experimental.pallas import tpu as pltpu
from jax.experimental.pallas import tpu_sc as plsc

NN = 10000
DD = 128
GG = 64
EE = 160000
NP = 10240
NR = NP // 128
NC, NS = 2, 16
NW = NC * NS
CH = 8
NCH = 10
TSTEPS = NCH * CH
EP = NW * TSTEPS * 128
TRASH = NN
RB = 1024
GRID = NP // RB
RPT = NP // NS
GPT = GG // NW
WPOOL = 64

_Z16 = functools.partial(jnp.zeros, (16,), jnp.float32)


def _deg_body(dst_hbm, out_hbm, didx, ones, zbuf, scnt, sem):
    cid = lax.axis_index("c")
    sid = lax.axis_index("s")
    wid = sid * NC + cid
    pltpu.sync_copy(dst_hbm.at[wid], didx)

    def _fill_ones(i, carry):
        ones[pl.ds(i * 16, 16)] = jnp.ones((16,), jnp.float32)
        return carry
    lax.fori_loop(0, 8, _fill_ones, 0)

    def _fill_zero(i, carry):
        zbuf[pl.ds(i * 16, 16)] = _Z16()
        return carry
    lax.fori_loop(0, 64, _fill_zero, 0)
    pltpu.sync_copy(zbuf, scnt.at[pl.ds(sid * 1024, 1024)])
    plsc.subcore_barrier()

    def _chunk(p, carry):
        for k in range(CH):
            pltpu.async_copy(ones, scnt.at[didx.at[p, k]], sem, add=True)
        for k in range(CH):
            pltpu.make_async_copy(ones, scnt.at[didx.at[0, 0]], sem).wait()
        return carry
    lax.fori_loop(0, NCH, _chunk, 0)
    plsc.subcore_barrier()
    pltpu.sync_copy(scnt.at[pl.ds(sid * 1024, 1024)],
                    out_hbm.at[cid, pl.ds(sid * 1024, 1024)])


def _scatter_body(y_hbm, src_hbm, dst_hbm, out_hbm,
                  sidx, didx, rows, acc, sem0, sem1, isem):
    cid = lax.axis_index("c")
    sid = lax.axis_index("s")
    wid = sid * NC + cid
    pltpu.async_copy(src_hbm.at[wid, 0], sidx.at[0], isem)
    pltpu.async_copy(dst_hbm.at[wid, 0], didx.at[0], isem)

    def _zero_rows(i, carry):
        for k in range(8):
            rows[0, i, pl.ds(k * 16, 16)] = _Z16()
        return carry
    lax.fori_loop(0, 128, _zero_rows, 0)
    base = sid * RPT
    for k in range(RPT // 128):
        pltpu.sync_copy(rows.at[0], acc.at[pl.ds(base + k * 128, 128)])
    plsc.subcore_barrier()

    def _chunk(c, carry):
        slot = lax.rem(c, 2)
        pltpu.make_async_copy(src_hbm.at[wid, 0], sidx.at[0], isem).wait()
        pltpu.make_async_copy(dst_hbm.at[wid, 0], didx.at[0], isem).wait()

        @pl.when(c + 1 < NCH)
        def _():
            pltpu.async_copy(src_hbm.at[wid, c + 1], sidx.at[1 - slot], isem)
            pltpu.async_copy(dst_hbm.at[wid, c + 1], didx.at[1 - slot], isem)

        pltpu.async_copy(y_hbm.at[sidx.at[slot, 0]], rows.at[0], sem0)
        for k in range(CH):
            r = k % 2
            sem = sem0 if r == 0 else sem1
            if k + 1 < CH:
                nsem = sem1 if r == 0 else sem0
                pltpu.async_copy(
                    y_hbm.at[sidx.at[slot, k + 1]], rows.at[1 - r], nsem)
            pltpu.make_async_copy(y_hbm.at[sidx.at[slot, k]], rows.at[r], sem).wait()
            pltpu.sync_copy(rows.at[r], acc.at[didx.at[slot, k]], add=True)
        return carry
    lax.fori_loop(0, NCH, _chunk, 0)
    plsc.subcore_barrier()
    pltpu.sync_copy(acc.at[pl.ds(base, RPT)], out_hbm.at[cid, pl.ds(base, RPT)])


def _pool_body(h3_hbm, se_hbm, out_hbm, sebuf, win, accb):
    cid = lax.axis_index("c")
    sid = lax.axis_index("s")
    wid = sid * NC + cid
    pltpu.sync_copy(se_hbm, sebuf)
    for gi in range(GPT):
        for k in range(8):
            accb[gi, pl.ds(k * 16, 16)] = _Z16()
    for gi in range(GPT):
        g = wid * GPT + gi
        sev = sebuf[pl.ds(g, 16)]
        s = sev[0]
        e = sev[1]
        s0 = (s // 8) * 8

        def _window(w, carry):
            base = pl.multiple_of(s0 + w * WPOOL, 8)
            pltpu.sync_copy(h3_hbm.at[pl.ds(base, WPOOL)], win)

            def _row(j, c2):
                @pl.when(jnp.logical_and(base + j >= s, base + j < e))
                def _():
                    for k in range(8):
                        accb[gi, pl.ds(k * 16, 16)] = (
                            accb[gi, pl.ds(k * 16, 16)]
                            + win[j, pl.ds(k * 16, 16)])
                return c2
            lax.fori_loop(0, WPOOL, _row, 0, unroll=False)
            return carry
        lax.fori_loop(0, lax.div(e - s0 + (WPOOL - 1), WPOOL), _window, 0,
                      unroll=False)
    pltpu.sync_copy(accb, out_hbm.at[wid, pl.ds(0, GPT)])


@functools.lru_cache(maxsize=None)
def _sc_kernels():
    mesh = plsc.VectorSubcoreMesh(
        core_axis_name="c", subcore_axis_name="s",
        num_cores=NC, num_subcores=NS)
    deg = pl.kernel(
        _deg_body,
        out_type=jax.ShapeDtypeStruct((NC, 16384), jnp.float32),
        mesh=mesh,
        scratch_types=[
            pltpu.VMEM((NCH, CH, 128), jnp.int32),
            pltpu.VMEM((128,), jnp.float32),
            pltpu.VMEM((1024,), jnp.float32),
            pltpu.VMEM_SHARED((16384,), jnp.float32),
            pltpu.SemaphoreType.DMA,
        ],
    )
    scatter = pl.kernel(
        _scatter_body,
        out_type=jax.ShapeDtypeStruct((NC, NP, DD), jnp.float32),
        mesh=mesh,
        scratch_types=[
            pltpu.VMEM((2, CH, 128), jnp.int32),
            pltpu.VMEM((2, CH, 128), jnp.int32),
            pltpu.VMEM((2, 128, DD), jnp.float32),
            pltpu.VMEM_SHARED((NP, DD), jnp.float32),
            pltpu.SemaphoreType.DMA,
            pltpu.SemaphoreType.DMA,
            pltpu.SemaphoreType.DMA,
        ],
    )
    pool = pl.kernel(
        _pool_body,
        out_type=jax.ShapeDtypeStruct((NW, 8, DD), jnp.float32),
        mesh=mesh,
        scratch_types=[
            pltpu.VMEM((128,), jnp.int32),
            pltpu.VMEM((WPOOL, DD), jnp.float32),
            pltpu.VMEM((GPT, DD), jnp.float32),
        ],
    )
    return deg, scatter, pool


def _scale_rows(v, dinv):
    return (v.reshape(RB // 128, 128, DD) * dinv[:, :, None]).reshape(RB, DD)


def _tc_pre_body(cnt_ref, x_ref, w_ref, dinv_ref, y_ref):
    deg = jnp.sum(cnt_ref[...], axis=0) + 1.0
    dinv = lax.rsqrt(deg)
    dinv_ref[...] = dinv
    xw = jnp.dot(x_ref[...], w_ref[...], preferred_element_type=jnp.float32)
    y_ref[...] = _scale_rows(xw, dinv)


_pre_call = pl.pallas_call(
    _tc_pre_body,
    grid=(GRID,),
    in_specs=[
        pl.BlockSpec((NC, RB // 128, 128), lambda i: (0, i, 0)),
        pl.BlockSpec((RB, DD), lambda i: (i, 0)),
        pl.BlockSpec((DD, DD), lambda i: (0, 0)),
    ],
    out_specs=[
        pl.BlockSpec((RB // 128, 128), lambda i: (i, 0)),
        pl.BlockSpec((RB, DD), lambda i: (i, 0)),
    ],
    out_shape=[
        jax.ShapeDtypeStruct((NR, 128), jnp.float32),
        jax.ShapeDtypeStruct((NP, DD), jnp.float32),
    ],
)


def _tc_mid_body(acc_ref, y_ref, dinv_ref, b_ref, w_ref, ynext_ref):
    dinv = dinv_ref[...]
    s = acc_ref[0] + acc_ref[1] + y_ref[...]
    h = jnp.maximum(_scale_rows(s, dinv) + b_ref[...], 0.0)
    hw = jnp.dot(h, w_ref[...], preferred_element_type=jnp.float32)
    ynext_ref[...] = _scale_rows(hw, dinv)


_mid_call = pl.pallas_call(
    _tc_mid_body,
    grid=(GRID,),
    in_specs=[
        pl.BlockSpec((2, RB, DD), lambda i: (0, i, 0)),
        pl.BlockSpec((RB, DD), lambda i: (i, 0)),
        pl.BlockSpec((RB // 128, 128), lambda i: (i, 0)),
        pl.BlockSpec((1, DD), lambda i: (0, 0)),
        pl.BlockSpec((DD, DD), lambda i: (0, 0)),
    ],
    out_specs=pl.BlockSpec((RB, DD), lambda i: (i, 0)),
    out_shape=jax.ShapeDtypeStruct((NP, DD), jnp.float32),
)


def _bn(z, g, b):
    mu = jnp.mean(z, axis=0, keepdims=True)
    var = jnp.mean((z - mu) ** 2, axis=0, keepdims=True)
    return (z - mu) / jnp.sqrt(var + 1e-5) * g + b


def _lrelu(z):
    return jnp.where(z > 0, z, 0.01 * z)


def _tc_h3_body(acc_ref, y_ref, dinv_ref, b_ref, batch_ref,
                h3_ref, se_ref, cntg):
    i = pl.program_id(0)

    @pl.when(i == 0)
    def _():
        cntg[...] = jnp.zeros((1, GG), jnp.float32)

    dinv = dinv_ref[...]
    s = acc_ref[0] + acc_ref[1] + y_ref[...]
    h3_ref[...] = _scale_rows(s, dinv) + b_ref[...]
    bt = batch_ref[...]
    gid = lax.broadcasted_iota(jnp.int32, (1, 1, GG), 2)
    pt = (bt[:, :, None] == gid).astype(jnp.float32).reshape(RB, GG)
    ones_row = jnp.ones((1, RB), jnp.float32)
    cntg[...] += jnp.dot(ones_row, pt, preferred_element_type=jnp.float32,
                         precision=lax.Precision.HIGHEST)

    @pl.when(i == GRID - 1)
    def _():
        gi2 = lax.broadcasted_iota(jnp.int32, (GG, 128), 0)
        gj2 = lax.broadcasted_iota(jnp.int32, (GG, 128), 1)
        ut = (gi2 < gj2).astype(jnp.float32)
        starts = jnp.dot(cntg[...], ut, preferred_element_type=jnp.float32,
                         precision=lax.Precision.HIGHEST)
        se_ref[...] = starts.astype(jnp.int32)


_h3_call = pl.pallas_call(
    _tc_h3_body,
    grid=(GRID,),
    in_specs=[
        pl.BlockSpec((2, RB, DD), lambda i: (0, i, 0)),
        pl.BlockSpec((RB, DD), lambda i: (i, 0)),
        pl.BlockSpec((RB // 128, 128), lambda i: (i, 0)),
        pl.BlockSpec((1, DD), lambda i: (0, 0)),
        pl.BlockSpec((RB // 128, 128), lambda i: (i, 0)),
    ],
    out_specs=[
        pl.BlockSpec((RB, DD), lambda i: (i, 0)),
        pl.BlockSpec((1, 128), lambda i: (0, 0)),
    ],
    out_shape=[
        jax.ShapeDtypeStruct((NP, DD), jnp.float32),
        jax.ShapeDtypeStruct((1, 128), jnp.int32),
    ],
    scratch_shapes=[pltpu.VMEM((1, GG), jnp.float32)],
)


def _tc_head_body(p_ref, fw1, fb1, g1, be1, fw2, fb2, g2, be2, fw3, fb3,
                  out_ref):
    p = p_ref[...]
    z = jnp.dot(p, fw1[...], preferred_element_type=jnp.float32) + fb1[...]
    z = _bn(_lrelu(z), g1[...], be1[...])
    z = jnp.dot(z, fw2[...], preferred_element_type=jnp.float32) + fb2[...]
    z = _bn(_lrelu(z), g2[...], be2[...])
    out_ref[...] = jnp.dot(z, fw3[...].reshape(DD, 1),
                           preferred_element_type=jnp.float32) + fb3[...]


_head_call = pl.pallas_call(
    _tc_head_body,
    in_specs=[pl.BlockSpec((GG, DD), lambda: (0, 0)),
              pl.BlockSpec((DD, DD), lambda: (0, 0)),
              pl.BlockSpec((1, DD), lambda: (0, 0)),
              pl.BlockSpec((1, DD), lambda: (0, 0)),
              pl.BlockSpec((1, DD), lambda: (0, 0)),
              pl.BlockSpec((DD, DD), lambda: (0, 0)),
              pl.BlockSpec((1, DD), lambda: (0, 0)),
              pl.BlockSpec((1, DD), lambda: (0, 0)),
              pl.BlockSpec((1, DD), lambda: (0, 0)),
              pl.BlockSpec((1, DD), lambda: (0, 0)),
              pl.BlockSpec((1, 1), lambda: (0, 0))],
    out_specs=pl.BlockSpec((GG, 1), lambda: (0, 0)),
    out_shape=jax.ShapeDtypeStruct((GG, 1), jnp.float32),
)


def kernel(x, edge_index_intra, edge_index_inter, batch,
           W1, b1, W2, b2, W3, b3,
           fW1, fb1, g1, be1, fW2, fb2, g2, be2, fW3, fb3):
    src = jnp.concatenate([edge_index_intra[0], edge_index_inter[0]])
    dst = jnp.concatenate([edge_index_intra[1], edge_index_inter[1]])
    pad = (jnp.arange(EP - 2 * EE, dtype=jnp.int32) % (NP - NN)) + NN
    srcp = jnp.concatenate([src, pad]).reshape(NW, NCH, CH, 128)
    dstp = jnp.concatenate([dst, pad]).reshape(NW, NCH, CH, 128)
    xp = jnp.pad(x, ((0, NP - NN), (0, 0)))
    batchp = jnp.pad(batch, (0, NP - NN), constant_values=GG).reshape(NR, 128)

    deg_call, scatter_call, pool_call = _sc_kernels()
    cnt = deg_call(dstp)
    cnt2 = cnt[:, :NP].reshape(NC, NR, 128)

    dinv, y1 = _pre_call(cnt2, xp, W1)
    acc1 = scatter_call(y1, srcp, dstp)
    y2 = _mid_call(acc1, y1, dinv, b1.reshape(1, DD), W2)
    acc2 = scatter_call(y2, srcp, dstp)
    y3 = _mid_call(acc2, y2, dinv, b2.reshape(1, DD), W3)
    acc3 = scatter_call(y3, srcp, dstp)

    h3, se = _h3_call(acc3, y3, dinv, b3.reshape(1, DD), batchp)
    pooled = pool_call(h3, se.reshape(128))
    out = _head_call(
        pooled[:, :GPT].reshape(GG, DD),
        fW1, fb1.reshape(1, DD), g1.reshape(1, DD), be1.reshape(1, DD),
        fW2, fb2.reshape(1, DD), g2.reshape(1, DD), be2.reshape(1, DD),
        fW3.reshape(1, DD), fb3.reshape(1, 1))
    return out.reshape(-1)
